# baseline (speedup 1.0000x reference)
import functools

import jax
import jax.numpy as jnp
from jax import lax
from jax.experimental import pallas as pl
from jax.experimental.pallas import tpu as pltpu
from jax.experimental.pallas import tpu_sc as plsc

VOCAB = 1000000
EMB = 32
B = 4096
L = 200
PACK = 4
ROWS = VOCAB // PACK
NC, NS = 2, 16
NW = NC * NS
BBLK = B // NW
NBUF = 2

_mesh = plsc.VectorSubcoreMesh(core_axis_name="c", subcore_axis_name="s")


@functools.partial(
    pl.kernel,
    out_type=jax.ShapeDtypeStruct((L, EMB, B), jnp.float32),
    mesh=_mesh,
    scratch_types=(
        [
            pltpu.VMEM((L, BBLK), jnp.int32),
            pltpu.VMEM((L, BBLK), jnp.int32),
            pltpu.VMEM((NBUF, BBLK, 128), jnp.float32),
            pltpu.VMEM((NBUF, EMB, BBLK), jnp.float32),
        ]
        + [pltpu.SemaphoreType.DMA] * (2 * NBUF)
    ),
    compiler_params=pltpu.CompilerParams(
        use_tc_tiling_on_sc=True, needs_layout_passes=False
    ),
)
def _embed_kernel(idxt_hbm, tab4_hbm, out_hbm, idx_v, row_v, buf, oblk, *sems):
    gsem = sems[:NBUF]
    wsem = sems[NBUF:]
    wid = lax.axis_index("s") * NC + lax.axis_index("c")
    b0 = wid * BBLK

    pltpu.sync_copy(idxt_hbm.at[pl.ds(0, L), pl.ds(b0, BBLK)], idx_v)

    lanes = [lax.iota(jnp.int32, 16) + 16 * g for g in range(BBLK // 16)]

    @pl.loop(0, L)
    def _prep(l):
        for g in range(BBLK // 16):
            v = idx_v[l, pl.ds(16 * g, 16)]
            row_v[l, pl.ds(16 * g, 16)] = lax.shift_right_logical(v, 2)
            idx_v[l, pl.ds(16 * g, 16)] = lax.shift_left(jnp.bitwise_and(v, 3), 5)

    def gather(l, s):
        return pltpu.make_async_copy(
            tab4_hbm.at[row_v.at[l]], buf.at[s], gsem[s]
        )

    def writeback(l, s):
        return pltpu.make_async_copy(
            oblk.at[s], out_hbm.at[l, pl.ds(0, EMB), pl.ds(b0, BBLK)], wsem[s]
        )

    def extract(l, s):
        for g in range(BBLK // 16):
            rem32 = idx_v[l, pl.ds(16 * g, 16)]
            lane = lanes[g]
            for e in range(EMB):
                val = plsc.load_gather(buf.at[s], [lane, rem32 + e])
                oblk[s, e, pl.ds(16 * g, 16)] = val

    for s in range(NBUF):
        gather(s, s).start()

    @pl.loop(0, L - NBUF, step=NBUF)
    def _main(i):
        for s in range(NBUF):
            l = i + s
            gather(l, s).wait()

            @pl.when(l >= NBUF)
            def _():
                writeback(l - NBUF, s).wait()

            extract(l, s)
            writeback(l, s).start()
            gather(l + NBUF, s).start()

    for s in range(NBUF):
        l = L - NBUF + s
        gather(l, s).wait()
        writeback(l - NBUF, s).wait()
        extract(l, s)
        writeback(l, s).start()
    for s in range(NBUF):
        writeback(L - NBUF + s, s).wait()


def kernel(inputs, table):
    idxt = inputs.astype(jnp.int32).T
    tab4 = table.reshape(ROWS, PACK * EMB)
    out = _embed_kernel(idxt, tab4)
    return out.transpose(2, 0, 1)


# speedup vs baseline: 1.3255x; 1.3255x over previous
import functools

import jax
import jax.numpy as jnp
from jax import lax
from jax.experimental import pallas as pl
from jax.experimental.pallas import tpu as pltpu
from jax.experimental.pallas import tpu_sc as plsc

VOCAB = 1000000
EMB = 32
B = 4096
L = 200
PACK = 4
ROWS = VOCAB // PACK
NC, NS = 2, 16
NW = NC * NS
BBLK = B // NW
NBUF = 4
NWB = 2

_mesh = plsc.VectorSubcoreMesh(core_axis_name="c", subcore_axis_name="s")


@functools.partial(
    pl.kernel,
    out_type=jax.ShapeDtypeStruct((L, EMB, B), jnp.float32),
    mesh=_mesh,
    scratch_types=(
        [
            pltpu.VMEM((L, BBLK), jnp.int32),
            pltpu.VMEM((L, BBLK), jnp.int32),
            pltpu.VMEM((NBUF, BBLK, 128), jnp.float32),
            pltpu.VMEM((NWB, EMB, BBLK), jnp.float32),
        ]
        + [pltpu.SemaphoreType.DMA] * (NBUF + NWB)
    ),
    compiler_params=pltpu.CompilerParams(
        use_tc_tiling_on_sc=True, needs_layout_passes=False
    ),
)
def _embed_kernel(idxt_hbm, tab4_hbm, out_hbm, idx_v, row_v, buf, oblk, *sems):
    gsem = sems[:NBUF]
    wsem = sems[NBUF:]
    wid = lax.axis_index("s") * NC + lax.axis_index("c")
    b0 = wid * BBLK

    pltpu.sync_copy(idxt_hbm.at[pl.ds(0, L), pl.ds(b0, BBLK)], idx_v)

    lanes = [lax.iota(jnp.int32, 16) + 16 * g for g in range(BBLK // 16)]

    @pl.loop(0, L)
    def _prep(l):
        for g in range(BBLK // 16):
            v = idx_v[l, pl.ds(16 * g, 16)]
            row_v[l, pl.ds(16 * g, 16)] = lax.shift_right_logical(v, 2)
            idx_v[l, pl.ds(16 * g, 16)] = lax.shift_left(jnp.bitwise_and(v, 3), 5)

    def gather(l, s):
        return pltpu.make_async_copy(
            tab4_hbm.at[row_v.at[l]], buf.at[s], gsem[s]
        )

    def writeback(l, w):
        return pltpu.make_async_copy(
            oblk.at[w], out_hbm.at[l, pl.ds(0, EMB), pl.ds(b0, BBLK)], wsem[w]
        )

    def extract(l, s, w):
        @plsc.parallel_loop(0, BBLK // 16, unroll=2)
        def _grps(g):
            rem32 = idx_v[l, pl.ds(16 * g, 16)]
            lane = lax.iota(jnp.int32, 16) + 16 * g
            for e in range(EMB):
                val = plsc.load_gather(buf.at[s], [lane, rem32 + e])
                oblk[w, e, pl.ds(16 * g, 16)] = val

    for s in range(NBUF):
        gather(s, s).start()

    @pl.loop(0, L - NBUF, step=NBUF)
    def _main(i):
        for s in range(NBUF):
            l = i + s
            w = s % NWB
            gather(l, s).wait()

            @pl.when(l >= NWB)
            def _():
                writeback(l - NWB, w).wait()

            extract(l, s, w)
            writeback(l, w).start()
            gather(l + NBUF, s).start()

    for s in range(NBUF):
        l = L - NBUF + s
        w = s % NWB
        gather(l, s).wait()
        writeback(l - NWB, w).wait()
        extract(l, s, w)
        writeback(l, w).start()
    for w in range(NWB):
        writeback(L - NWB + w, w).wait()


def kernel(inputs, table):
    idxt = inputs.astype(jnp.int32).T
    tab4 = table.reshape(ROWS, PACK * EMB)
    out = _embed_kernel(idxt, tab4)
    return out.transpose(2, 0, 1)


# speedup vs baseline: 1.3599x; 1.0260x over previous
import functools

import jax
import jax.numpy as jnp
from jax import lax
from jax.experimental import pallas as pl
from jax.experimental.pallas import tpu as pltpu
from jax.experimental.pallas import tpu_sc as plsc

VOCAB = 1000000
EMB = 32
B = 4096
L = 200
PACK = 4
ROWS = VOCAB // PACK
NC, NS = 2, 16
NW = NC * NS
BBLK = B // NW
NBUF = 4
NWB = 2

_mesh = plsc.VectorSubcoreMesh(core_axis_name="c", subcore_axis_name="s")


@functools.partial(
    pl.kernel,
    out_type=jax.ShapeDtypeStruct((L, EMB, B), jnp.float32),
    mesh=_mesh,
    scratch_types=(
        [
            pltpu.VMEM((L, BBLK), jnp.int32),
            pltpu.VMEM((L, BBLK), jnp.int32),
            pltpu.VMEM((NBUF, BBLK, 128), jnp.float32),
            pltpu.VMEM((NWB, EMB, BBLK), jnp.float32),
        ]
        + [pltpu.SemaphoreType.DMA] * (NBUF + NWB)
    ),
    compiler_params=pltpu.CompilerParams(
        use_tc_tiling_on_sc=True, needs_layout_passes=False
    ),
)
def _embed_kernel(idxt_hbm, tab4_hbm, out_hbm, idx_v, row_v, buf, oblk, *sems):
    gsem = sems[:NBUF]
    wsem = sems[NBUF:]
    wid = lax.axis_index("s") * NC + lax.axis_index("c")
    b0 = wid * BBLK

    pltpu.sync_copy(idxt_hbm.at[pl.ds(0, L), pl.ds(b0, BBLK)], idx_v)

    lanes = [lax.iota(jnp.int32, 16) + 16 * g for g in range(BBLK // 16)]

    @pl.loop(0, L)
    def _prep(l):
        for g in range(BBLK // 16):
            v = idx_v[l, pl.ds(16 * g, 16)]
            row_v[l, pl.ds(16 * g, 16)] = lax.shift_right_logical(v, 2)
            idx_v[l, pl.ds(16 * g, 16)] = lax.shift_left(jnp.bitwise_and(v, 3), 5)

    def gather(l, s):
        return pltpu.make_async_copy(
            tab4_hbm.at[row_v.at[l]], buf.at[s], gsem[s]
        )

    def writeback(l, w):
        return pltpu.make_async_copy(
            oblk.at[w], out_hbm.at[l, pl.ds(0, EMB), pl.ds(b0, BBLK)], wsem[w]
        )

    def extract(l, s, w):
        for g in range(BBLK // 16):
            rem32 = idx_v[l, pl.ds(16 * g, 16)]
            lane = lanes[g]

            @plsc.parallel_loop(0, EMB, unroll=8)
            def _cols(e, _g=g, _lane=lane, _rem=rem32):
                val = plsc.load_gather(buf.at[s], [_lane, _rem + e])
                oblk[w, e, pl.ds(16 * _g, 16)] = val

    for s in range(NBUF):
        gather(s, s).start()

    @pl.loop(0, L - NBUF, step=NBUF)
    def _main(i):
        for s in range(NBUF):
            l = i + s
            w = s % NWB
            gather(l, s).wait()

            @pl.when(l >= NWB)
            def _():
                writeback(l - NWB, w).wait()

            extract(l, s, w)
            writeback(l, w).start()
            gather(l + NBUF, s).start()

    for s in range(NBUF):
        l = L - NBUF + s
        w = s % NWB
        gather(l, s).wait()
        writeback(l - NWB, w).wait()
        extract(l, s, w)
        writeback(l, w).start()
    for w in range(NWB):
        writeback(L - NWB + w, w).wait()


def kernel(inputs, table):
    idxt = inputs.astype(jnp.int32).T
    tab4 = table.reshape(ROWS, PACK * EMB)
    out = _embed_kernel(idxt, tab4)
    return out.transpose(2, 0, 1)
